# baseline (speedup 1.0000x reference)
import functools

import jax
import jax.numpy as jnp
import numpy as np
from jax import lax
from jax.experimental import pallas as pl
from jax.experimental.pallas import tpu as pltpu
from jax.experimental.pallas import tpu_sc as plsc

N_NODES = 10000
N_EVENTS = 640000
BATCH_NODES = 256
PENALTY = 10.0
TIME_MAX = 100.0

_step = 1.0 / 5
_CPS = np.asarray(np.arange(0.0, 1.0 + _step, _step) * (TIME_MAX + 0.0001),
                  dtype=np.float32)
N_CP = int(_CPS.shape[0])
N_SEG = N_CP - 1
SEG_LEN = float(_CPS[1] - _CPS[0])
SQRT2PI = float(np.sqrt(2.0 * np.pi))
INV_SQRT2 = float(1.0 / np.sqrt(2.0))


NC, NS, L = 2, 16, 16
NW = NC * NS
EPW = N_EVENTS // NW
CHUNK = 800
N_CHUNKS = EPW // CHUNK
ZWORDS = N_NODES * 2 * N_CP

def _sc_events_body(ts_hbm, s_hbm, r_hbm, z_hbm, out_hbm,
                    z_v, ts_v, s_v, r_v, acc_v):
    wid = lax.axis_index("s") * NC + lax.axis_index("c")
    pltpu.sync_copy(z_hbm, z_v)
    base = wid * EPW
    inv_seg = jnp.float32(1.0 / SEG_LEN)

    def chunk_body(ci, acc):
        off = base + ci * CHUNK
        pltpu.sync_copy(ts_hbm.at[pl.ds(off, CHUNK)], ts_v)
        pltpu.sync_copy(s_hbm.at[pl.ds(off, CHUNK)], s_v)
        pltpu.sync_copy(r_hbm.at[pl.ds(off, CHUNK)], r_v)

        def ev_body(i, a):
            sl = pl.ds(i * L, L)
            ts = ts_v[sl]
            sn = s_v[sl]
            rn = r_v[sl]
            x = ts * inv_seg
            kap = jnp.minimum(x.astype(jnp.int32), N_CP - 2)
            delta = x - kap.astype(jnp.float32)
            omd = 1.0 - delta
            sb = sn * (2 * N_CP) + kap
            rb = rn * (2 * N_CP) + kap
            sxc = plsc.load_gather(z_v, [sb])
            sxn = plsc.load_gather(z_v, [sb + 1])
            syc = plsc.load_gather(z_v, [sb + N_CP])
            syn = plsc.load_gather(z_v, [sb + (N_CP + 1)])
            rxc = plsc.load_gather(z_v, [rb])
            rxn = plsc.load_gather(z_v, [rb + 1])
            ryc = plsc.load_gather(z_v, [rb + N_CP])
            ryn = plsc.load_gather(z_v, [rb + (N_CP + 1)])
            dx = omd * (sxc - rxc) + delta * (sxn - rxn)
            dy = omd * (syc - ryc) + delta * (syn - ryn)
            return a + dx * dx + dy * dy

        return lax.fori_loop(0, CHUNK // L, ev_body, acc)

    acc = lax.fori_loop(0, N_CHUNKS, chunk_body, jnp.zeros((L,), jnp.float32))
    acc_v[...] = acc
    pltpu.sync_copy(acc_v, out_hbm.at[wid])


@functools.cache
def _sc_events_kernel():
    mesh = plsc.VectorSubcoreMesh(core_axis_name="c", subcore_axis_name="s",
                                  num_cores=NC, num_subcores=NS)
    return pl.kernel(
        _sc_events_body,
        mesh=mesh,
        out_type=jax.ShapeDtypeStruct((NW, L), jnp.float32),
        scratch_types=[
            pltpu.VMEM((ZWORDS,), jnp.float32),
            pltpu.VMEM((CHUNK,), jnp.float32),
            pltpu.VMEM((CHUNK,), jnp.int32),
            pltpu.VMEM((CHUNK,), jnp.int32),
            pltpu.VMEM((L,), jnp.float32),
        ],
        compiler_params=pltpu.CompilerParams(needs_layout_passes=False),
    )


BJ = 2000
NJ = N_NODES // BJ


def _phi(x):
    return 0.5 * (1.0 + lax.erf(x * INV_SQRT2))


def _tc_body(zn_ref, z_ref, nodes_ref, out_ref, zt_s):
    j = pl.program_id(0)
    k = pl.program_id(1)

    @pl.when((k == 0) & (j == 0))
    def _():
        out_ref[...] = jnp.zeros((1, 2), jnp.float32)

    @pl.when(k == 0)
    def _():
        zt_s[...] = jnp.transpose(z_ref[...], (1, 0))

    znx1 = zn_ref[0, 0]
    zny1 = zn_ref[0, 1]
    znx2 = zn_ref[0, 2]
    zny2 = zn_ref[0, 3]
    zx1 = zt_s[pl.ds(k, 1), :]
    zy1 = zt_s[pl.ds(k + N_CP, 1), :]
    zx2 = zt_s[pl.ds(k + 1, 1), :]
    zy2 = zt_s[pl.ds(k + N_CP + 1, 1), :]

    d1x = znx1 - zx1
    d1y = zny1 - zy1
    ddx = (znx2 - zx2) - d1x
    ddy = (zny2 - zy2) - d1y
    a = ddx * ddx + ddy * ddy + 1e-12
    b = 2.0 * (d1x * ddx + d1y * ddy)
    c = d1x * d1x + d1y * d1y
    mu = -b / (2.0 * a)
    rs = jnp.sqrt(2.0 * a)
    sigma = 1.0 / rs
    peak = jnp.exp(a * mu * mu - c)
    seg = peak * sigma * SQRT2PI * (_phi((1.0 - mu) * rs) - _phi(-mu * rs))

    col = j * BJ + lax.broadcasted_iota(jnp.int32, (BATCH_NODES, BJ), 1)
    nb = nodes_ref[...]
    valid = col != nb
    contrib = (SEG_LEN * 0.5) * jnp.sum(jnp.where(valid, seg, 0.0))

    dpx = znx2 - znx1
    dpy = zny2 - zny1
    pk = (PENALTY * jnp.sum(dpx * dpx + dpy * dpy)
          * jnp.where(j == 0, 1.0, 0.0))
    lane = lax.broadcasted_iota(jnp.int32, (1, 2), 1)
    out_ref[...] += jnp.where(lane == 0, pk, contrib)


def _tc_call(zn4, zr, nodes2):
    return pl.pallas_call(
        _tc_body,
        grid=(NJ, N_SEG),
        in_specs=[
            pl.BlockSpec((1, 4, BATCH_NODES, 1), lambda j, k: (k, 0, 0, 0)),
            pl.BlockSpec((BJ, 2 * N_CP), lambda j, k: (j, 0)),
            pl.BlockSpec((BATCH_NODES, 1), lambda j, k: (0, 0)),
        ],
        out_specs=pl.BlockSpec((1, 2), lambda j, k: (0, 0)),
        out_shape=jax.ShapeDtypeStruct((1, 2), jnp.float32),
        scratch_shapes=[pltpu.VMEM((2 * N_CP, BJ), jnp.float32)],
    )(zn4, zr, nodes2)


_ZN_COLS = np.array([[k, N_CP + k, k + 1, N_CP + k + 1] for k in range(N_SEG)],
                    dtype=np.int32)


def kernel(timestamps, senders, receivers, nodes, Z, beta):
    z_flat = Z.reshape(ZWORDS)
    zr = Z.reshape(N_NODES, 2 * N_CP)

    zn4 = zr[nodes[None, None, :], jnp.asarray(_ZN_COLS)[:, :, None]]
    zn4 = zn4.reshape(N_SEG, 4, BATCH_NODES, 1)

    nodes2 = nodes.reshape(BATCH_NODES, 1)

    sc_out = _sc_events_kernel()(timestamps, senders, receivers, z_flat)
    tc_out = _tc_call(zn4, zr, nodes2)

    s_events = jnp.sum(sc_out)
    prior = tc_out[0, 0]
    integral = jnp.exp(beta) * tc_out[0, 1]
    scale = float(N_NODES) / float(BATCH_NODES)
    return scale * (prior + integral + s_events - N_EVENTS * beta)


# speedup vs baseline: 1.0239x; 1.0239x over previous
import functools

import jax
import jax.numpy as jnp
import numpy as np
from jax import lax
from jax.experimental import pallas as pl
from jax.experimental.pallas import tpu as pltpu
from jax.experimental.pallas import tpu_sc as plsc

N_NODES = 10000
N_EVENTS = 640000
BATCH_NODES = 256
PENALTY = 10.0
TIME_MAX = 100.0

_step = 1.0 / 5
_CPS = np.asarray(np.arange(0.0, 1.0 + _step, _step) * (TIME_MAX + 0.0001),
                  dtype=np.float32)
N_CP = int(_CPS.shape[0])
N_SEG = N_CP - 1
SEG_LEN = float(_CPS[1] - _CPS[0])
SQRT2PI = float(np.sqrt(2.0 * np.pi))
INV_SQRT2 = float(1.0 / np.sqrt(2.0))


NC, NS, L = 2, 16, 16
NW = NC * NS
EPW = N_EVENTS // NW
CHUNK = 800
N_CHUNKS = EPW // CHUNK
ZWORDS = N_NODES * 2 * N_CP

def _sc_events_body(ts_hbm, s_hbm, r_hbm, z_hbm, out_hbm,
                    z_v, ts_v, s_v, r_v, acc_v):
    wid = lax.axis_index("s") * NC + lax.axis_index("c")
    pltpu.sync_copy(z_hbm, z_v)
    base = wid * EPW
    inv_seg = jnp.float32(1.0 / SEG_LEN)

    def chunk_body(ci, acc):
        off = base + ci * CHUNK
        pltpu.sync_copy(ts_hbm.at[pl.ds(off, CHUNK)], ts_v)
        pltpu.sync_copy(s_hbm.at[pl.ds(off, CHUNK)], s_v)
        pltpu.sync_copy(r_hbm.at[pl.ds(off, CHUNK)], r_v)

        def ev_body(i, a):
            sl = pl.ds(i * L, L)
            ts = ts_v[sl]
            sn = s_v[sl]
            rn = r_v[sl]
            x = ts * inv_seg
            kap = jnp.minimum(x.astype(jnp.int32), N_CP - 2)
            delta = x - kap.astype(jnp.float32)
            omd = 1.0 - delta
            sb = sn * (2 * N_CP) + kap
            rb = rn * (2 * N_CP) + kap
            sxc = plsc.load_gather(z_v, [sb])
            sxn = plsc.load_gather(z_v, [sb + 1])
            syc = plsc.load_gather(z_v, [sb + N_CP])
            syn = plsc.load_gather(z_v, [sb + (N_CP + 1)])
            rxc = plsc.load_gather(z_v, [rb])
            rxn = plsc.load_gather(z_v, [rb + 1])
            ryc = plsc.load_gather(z_v, [rb + N_CP])
            ryn = plsc.load_gather(z_v, [rb + (N_CP + 1)])
            dx = omd * (sxc - rxc) + delta * (sxn - rxn)
            dy = omd * (syc - ryc) + delta * (syn - ryn)
            return a + dx * dx + dy * dy

        return lax.fori_loop(0, CHUNK // L, ev_body, acc)

    acc = lax.fori_loop(0, N_CHUNKS, chunk_body, jnp.zeros((L,), jnp.float32))
    acc_v[...] = acc
    pltpu.sync_copy(acc_v, out_hbm.at[wid])


@functools.cache
def _sc_events_kernel():
    mesh = plsc.VectorSubcoreMesh(core_axis_name="c", subcore_axis_name="s",
                                  num_cores=NC, num_subcores=NS)
    return pl.kernel(
        _sc_events_body,
        mesh=mesh,
        out_type=jax.ShapeDtypeStruct((NW, L), jnp.float32),
        scratch_types=[
            pltpu.VMEM((ZWORDS,), jnp.float32),
            pltpu.VMEM((CHUNK,), jnp.float32),
            pltpu.VMEM((CHUNK,), jnp.int32),
            pltpu.VMEM((CHUNK,), jnp.int32),
            pltpu.VMEM((L,), jnp.float32),
        ],
        compiler_params=pltpu.CompilerParams(needs_layout_passes=False),
    )


BJ = 2048
FS_PAD = 10240
NJ = FS_PAD // BJ


def _phi(x):
    return 0.5 * (1.0 + lax.erf(x * INV_SQRT2))


def _tc_body(zn_ref, zx1_ref, zy1_ref, zx2_ref, zy2_ref, nodes_ref, out_ref):
    j = pl.program_id(0)
    k = pl.program_id(1)

    @pl.when((k == 0) & (j == 0))
    def _():
        out_ref[...] = jnp.zeros((1, 2), jnp.float32)

    znx1 = zn_ref[0, 0]
    zny1 = zn_ref[0, 1]
    znx2 = zn_ref[0, 2]
    zny2 = zn_ref[0, 3]
    zx1 = zx1_ref[0]
    zy1 = zy1_ref[0]
    zx2 = zx2_ref[0]
    zy2 = zy2_ref[0]

    d1x = znx1 - zx1
    d1y = zny1 - zy1
    ddx = (znx2 - zx2) - d1x
    ddy = (zny2 - zy2) - d1y
    a = ddx * ddx + ddy * ddy + 1e-12
    b = 2.0 * (d1x * ddx + d1y * ddy)
    c = d1x * d1x + d1y * d1y
    mu = -b / (2.0 * a)
    rs = jnp.sqrt(2.0 * a)
    sigma = 1.0 / rs
    peak = jnp.exp(a * mu * mu - c)
    seg = peak * sigma * SQRT2PI * (_phi((1.0 - mu) * rs) - _phi(-mu * rs))

    col = j * BJ + lax.broadcasted_iota(jnp.int32, (BATCH_NODES, BJ), 1)
    nb = nodes_ref[...]
    valid = (col != nb) & (col < N_NODES)
    contrib = (SEG_LEN * 0.5) * jnp.sum(jnp.where(valid, seg, 0.0))

    dpx = znx2 - znx1
    dpy = zny2 - zny1
    pk = (PENALTY * jnp.sum(dpx * dpx + dpy * dpy)
          * jnp.where(j == 0, 1.0, 0.0))
    lane = lax.broadcasted_iota(jnp.int32, (1, 2), 1)
    out_ref[...] += jnp.where(lane == 0, pk, contrib)


def _tc_call(zn4, zt6, nodes2):
    row = pl.BlockSpec
    return pl.pallas_call(
        _tc_body,
        grid=(NJ, N_SEG),
        in_specs=[
            pl.BlockSpec((1, 4, BATCH_NODES, 1), lambda j, k: (k, 0, 0, 0)),
            row((1, 1, BJ), lambda j, k: (k, 0, j)),
            row((1, 1, BJ), lambda j, k: (k + N_CP, 0, j)),
            row((1, 1, BJ), lambda j, k: (k + 1, 0, j)),
            row((1, 1, BJ), lambda j, k: (k + N_CP + 1, 0, j)),
            pl.BlockSpec((BATCH_NODES, 1), lambda j, k: (0, 0)),
        ],
        out_specs=pl.BlockSpec((1, 2), lambda j, k: (0, 0)),
        out_shape=jax.ShapeDtypeStruct((1, 2), jnp.float32),
    )(zn4, zt6, zt6, zt6, zt6, nodes2)


_ZN_COLS = np.array([[k, N_CP + k, k + 1, N_CP + k + 1] for k in range(N_SEG)],
                    dtype=np.int32)


def kernel(timestamps, senders, receivers, nodes, Z, beta):
    zr = Z.reshape(N_NODES, 2 * N_CP)
    z_flat = zr.reshape(ZWORDS)
    zt6 = jnp.pad(zr.T, ((0, 0), (0, FS_PAD - N_NODES)))
    zt6 = zt6.reshape(2 * N_CP, 1, FS_PAD)

    zn4 = zr[nodes[None, None, :], jnp.asarray(_ZN_COLS)[:, :, None]]
    zn4 = zn4.reshape(N_SEG, 4, BATCH_NODES, 1)

    nodes2 = nodes.reshape(BATCH_NODES, 1)

    sc_out = _sc_events_kernel()(timestamps, senders, receivers, z_flat)
    tc_out = _tc_call(zn4, zt6, nodes2)

    s_events = jnp.sum(sc_out)
    prior = tc_out[0, 0]
    integral = jnp.exp(beta) * tc_out[0, 1]
    scale = float(N_NODES) / float(BATCH_NODES)
    return scale * (prior + integral + s_events - N_EVENTS * beta)


# speedup vs baseline: 1.3433x; 1.3119x over previous
import functools

import jax
import jax.numpy as jnp
import numpy as np
from jax import lax
from jax.experimental import pallas as pl
from jax.experimental.pallas import tpu as pltpu
from jax.experimental.pallas import tpu_sc as plsc

N_NODES = 10000
N_EVENTS = 640000
BATCH_NODES = 256
PENALTY = 10.0
TIME_MAX = 100.0

_step = 1.0 / 5
_CPS = np.asarray(np.arange(0.0, 1.0 + _step, _step) * (TIME_MAX + 0.0001),
                  dtype=np.float32)
N_CP = int(_CPS.shape[0])
N_SEG = N_CP - 1
SEG_LEN = float(_CPS[1] - _CPS[0])
SQRT2PI = float(np.sqrt(2.0 * np.pi))
INV_SQRT2 = float(1.0 / np.sqrt(2.0))


NC, NS, L = 2, 16, 16
NW = NC * NS
EPW = N_EVENTS // NW
CHUNK = 800
N_CHUNKS = EPW // CHUNK
ZWORDS = N_NODES * 2 * N_CP

def _sc_events_body(ts_hbm, s_hbm, r_hbm, z_hbm, out_hbm,
                    z_v, ts_v, s_v, r_v, acc_v):
    wid = lax.axis_index("s") * NC + lax.axis_index("c")
    pltpu.sync_copy(z_hbm, z_v)
    base = wid * EPW
    inv_seg = jnp.float32(1.0 / SEG_LEN)

    def chunk_body(ci, acc):
        off = base + ci * CHUNK
        pltpu.sync_copy(ts_hbm.at[pl.ds(off, CHUNK)], ts_v)
        pltpu.sync_copy(s_hbm.at[pl.ds(off, CHUNK)], s_v)
        pltpu.sync_copy(r_hbm.at[pl.ds(off, CHUNK)], r_v)

        def ev_body(i, a):
            sl = pl.ds(i * L, L)
            ts = ts_v[sl]
            sn = s_v[sl]
            rn = r_v[sl]
            x = ts * inv_seg
            kap = jnp.minimum(x.astype(jnp.int32), N_CP - 2)
            delta = x - kap.astype(jnp.float32)
            omd = 1.0 - delta
            sb = sn * (2 * N_CP) + kap
            rb = rn * (2 * N_CP) + kap
            sxc = plsc.load_gather(z_v, [sb])
            sxn = plsc.load_gather(z_v, [sb + 1])
            syc = plsc.load_gather(z_v, [sb + N_CP])
            syn = plsc.load_gather(z_v, [sb + (N_CP + 1)])
            rxc = plsc.load_gather(z_v, [rb])
            rxn = plsc.load_gather(z_v, [rb + 1])
            ryc = plsc.load_gather(z_v, [rb + N_CP])
            ryn = plsc.load_gather(z_v, [rb + (N_CP + 1)])
            dx = omd * (sxc - rxc) + delta * (sxn - rxn)
            dy = omd * (syc - ryc) + delta * (syn - ryn)
            return a + dx * dx + dy * dy

        return lax.fori_loop(0, CHUNK // L, ev_body, acc)

    acc = lax.fori_loop(0, N_CHUNKS, chunk_body, jnp.zeros((L,), jnp.float32))
    acc_v[...] = acc
    pltpu.sync_copy(acc_v, out_hbm.at[wid])


@functools.cache
def _sc_events_kernel():
    mesh = plsc.VectorSubcoreMesh(core_axis_name="c", subcore_axis_name="s",
                                  num_cores=NC, num_subcores=NS)
    return pl.kernel(
        _sc_events_body,
        mesh=mesh,
        out_type=jax.ShapeDtypeStruct((NW, L), jnp.float32),
        scratch_types=[
            pltpu.VMEM((ZWORDS,), jnp.float32),
            pltpu.VMEM((CHUNK,), jnp.float32),
            pltpu.VMEM((CHUNK,), jnp.int32),
            pltpu.VMEM((CHUNK,), jnp.int32),
            pltpu.VMEM((L,), jnp.float32),
        ],
        compiler_params=pltpu.CompilerParams(needs_layout_passes=False),
    )


BJ = 1024
FS_PAD = 10240
NJ = FS_PAD // BJ


def _phi(x):
    return 0.5 * (1.0 + lax.erf(x * INV_SQRT2))


def _tc_body(zn_ref, zx1_ref, zy1_ref, zx2_ref, zy2_ref, nodes_ref, out_ref):
    j = pl.program_id(0)
    k = pl.program_id(1)

    @pl.when((k == 0) & (j == 0))
    def _():
        out_ref[...] = jnp.zeros((1, 2), jnp.float32)

    znx1 = zn_ref[0, 0]
    zny1 = zn_ref[0, 1]
    znx2 = zn_ref[0, 2]
    zny2 = zn_ref[0, 3]
    zx1 = zx1_ref[0]
    zy1 = zy1_ref[0]
    zx2 = zx2_ref[0]
    zy2 = zy2_ref[0]

    d1x = znx1 - zx1
    d1y = zny1 - zy1
    ddx = (znx2 - zx2) - d1x
    ddy = (zny2 - zy2) - d1y
    a = ddx * ddx + ddy * ddy + 1e-12
    b = 2.0 * (d1x * ddx + d1y * ddy)
    c = d1x * d1x + d1y * d1y
    mu = -b / (2.0 * a)
    rs = jnp.sqrt(2.0 * a)
    sigma = 1.0 / rs
    peak = jnp.exp(a * mu * mu - c)
    seg = peak * sigma * SQRT2PI * (_phi((1.0 - mu) * rs) - _phi(-mu * rs))

    col = j * BJ + lax.broadcasted_iota(jnp.int32, (BATCH_NODES, BJ), 1)
    nb = nodes_ref[...]
    valid = (col != nb) & (col < N_NODES)
    contrib = (SEG_LEN * 0.5) * jnp.sum(jnp.where(valid, seg, 0.0))

    dpx = znx2 - znx1
    dpy = zny2 - zny1
    pk = (PENALTY * jnp.sum(dpx * dpx + dpy * dpy)
          * jnp.where(j == 0, 1.0, 0.0))
    lane = lax.broadcasted_iota(jnp.int32, (1, 2), 1)
    out_ref[...] += jnp.where(lane == 0, pk, contrib)


def _tc_call(zn4, zt6, nodes2):
    row = pl.BlockSpec
    return pl.pallas_call(
        _tc_body,
        grid=(NJ, N_SEG),
        in_specs=[
            pl.BlockSpec((1, 4, BATCH_NODES, 1), lambda j, k: (k, 0, 0, 0)),
            row((1, 1, BJ), lambda j, k: (k, 0, j)),
            row((1, 1, BJ), lambda j, k: (k + N_CP, 0, j)),
            row((1, 1, BJ), lambda j, k: (k + 1, 0, j)),
            row((1, 1, BJ), lambda j, k: (k + N_CP + 1, 0, j)),
            pl.BlockSpec((BATCH_NODES, 1), lambda j, k: (0, 0)),
        ],
        out_specs=pl.BlockSpec((1, 2), lambda j, k: (0, 0)),
        out_shape=jax.ShapeDtypeStruct((1, 2), jnp.float32),
    )(zn4, zt6, zt6, zt6, zt6, nodes2)


_ZN_COLS = np.array([[k, N_CP + k, k + 1, N_CP + k + 1] for k in range(N_SEG)],
                    dtype=np.int32)


def kernel(timestamps, senders, receivers, nodes, Z, beta):
    zr = Z.reshape(N_NODES, 2 * N_CP)
    z_flat = zr.reshape(ZWORDS)
    zt6 = jnp.pad(zr.T, ((0, 0), (0, FS_PAD - N_NODES)))
    zt6 = zt6.reshape(2 * N_CP, 1, FS_PAD)

    zn4 = zr[nodes[None, None, :], jnp.asarray(_ZN_COLS)[:, :, None]]
    zn4 = zn4.reshape(N_SEG, 4, BATCH_NODES, 1)

    nodes2 = nodes.reshape(BATCH_NODES, 1)

    sc_out = _sc_events_kernel()(timestamps, senders, receivers, z_flat)
    tc_out = _tc_call(zn4, zt6, nodes2)

    s_events = jnp.sum(sc_out)
    prior = tc_out[0, 0]
    integral = jnp.exp(beta) * tc_out[0, 1]
    scale = float(N_NODES) / float(BATCH_NODES)
    return scale * (prior + integral + s_events - N_EVENTS * beta)
